# baseline (speedup 1.0000x reference)
import jax
import jax.numpy as jnp
from jax import lax
from jax.experimental import pallas as pl
from jax.experimental.pallas import tpu as pltpu
from jax.experimental.pallas import tpu_sc as plsc

_INFO = plsc.get_sparse_core_info()
_NC = _INFO.num_cores
_NS = _INFO.num_subcores
_L = _INFO.num_lanes
_NW = _NC * _NS

_BATCH = 4096
_NNEG = 50
_D = 128
_CH = _D // _L
_BPW = _BATCH // _NW
_RPC = 4
_NCHUNK = _BPW // _RPC
_ROWS = _RPC * _NNEG
_NBUF = 3
_JG = 4
_OPAD = _JG * _L


def _body(aid_hbm, pid_hbm, nidf_hbm, users_hbm, items_hbm, out_hbm,
          aidx_v, pidx_v, nidx_v, a_v, bufs, pd_s, out_v, sems):
    wid = lax.axis_index("s") * _NC + lax.axis_index("c")
    base = wid * _BPW

    pltpu.sync_copy(aid_hbm.at[pl.ds(base, _BPW)], aidx_v)
    pltpu.sync_copy(pid_hbm.at[pl.ds(base, _BPW)], pidx_v)
    pltpu.sync_copy(nidf_hbm.at[pl.ds(base * _NNEG, _BPW * _NNEG)], nidx_v)

    def fire(ch, b):
        pltpu.async_copy(
            items_hbm.at[nidx_v.at[pl.ds(ch * _ROWS, _ROWS)]],
            bufs[b], sems[b])

    def drain(b):
        pltpu.make_async_copy(
            items_hbm.at[nidx_v.at[pl.ds(0, _ROWS)]], bufs[b], sems[b]).wait()

    cp_a = pltpu.async_copy(users_hbm.at[aidx_v], a_v, sems[_NBUF])
    cp_p = pltpu.async_copy(
        items_hbm.at[pidx_v], bufs[2].at[pl.ds(0, _BPW)], sems[2])
    fire(0, 0)
    fire(1, 1)

    i0 = lax.iota(jnp.int32, _L)
    perms = [i0 ^ 8, i0 ^ 4, i0 ^ 2, i0 ^ 1]

    def butterfly(acc):
        for p in perms:
            acc = acc + acc.at[p].get(mode="promise_in_bounds")
        return acc

    cp_a.wait()
    cp_p.wait()

    def prow(r, carry):
        acc = jnp.zeros((_L,), jnp.float32)
        for c in range(_CH):
            a = a_v[r, pl.ds(c * _L, _L)]
            b = bufs[2][r, pl.ds(c * _L, _L)]
            d = a - b
            acc = acc + d * d
        pd_s[r] = butterfly(acc)[0]
        return carry

    lax.fori_loop(0, _BPW, prow, 0)
    fire(2, 2)

    def row_compute(buf, ch, rloc):
        r = ch * _RPC + rloc
        a = [a_v[r, pl.ds(c * _L, _L)] for c in range(_CH)]
        pd = pd_s[r]
        for jg in range(_JG):
            njl = _L if (jg + 1) * _L <= _NNEG else _NNEG - jg * _L
            res = jnp.zeros((_L,), jnp.float32)
            for jl in range(njl):
                jj = jg * _L + jl
                acc = jnp.zeros((_L,), jnp.float32)
                for c in range(_CH):
                    b = buf[rloc * _NNEG + jj, pl.ds(c * _L, _L)]
                    d = a[c] - b
                    acc = acc + d * d
                res = jnp.where(i0 == jl, butterfly(acc), res)
            out_v[r, pl.ds(jg * _L, _L)] = pd - res

    def chunk_compute(b, ch):
        def rbody(rhalf, carry):
            for rsub in range(2):
                row_compute(bufs[b], ch, rhalf * 2 + rsub)
            return carry
        lax.fori_loop(0, _RPC // 2, rbody, 0)

    def trio(m, carry):
        ch0 = m * _NBUF
        for b in range(_NBUF):
            drain(b)
            chunk_compute(b, ch0 + b)

            @pl.when(ch0 + b + _NBUF < _NCHUNK)
            def _():
                fire(ch0 + b + _NBUF, b)

        return carry

    lax.fori_loop(0, _NCHUNK // _NBUF, trio, 0)
    for b in range(_NCHUNK // _NBUF * _NBUF, _NCHUNK):
        bb = b % _NBUF
        drain(bb)
        chunk_compute(bb, b)

    pltpu.sync_copy(out_v, out_hbm.at[pl.ds(base, _BPW)])


@jax.jit
def _pairwise_sc(anchor_ids, pos_ids, negf_ids, users, items):
    mesh = plsc.VectorSubcoreMesh(core_axis_name="c", subcore_axis_name="s")

    def body(aid, pid, nid, u, it, out, aidx, pidx, nidx, a_v,
             b0, b1, b2, pd_s, out_v, s0, s1, s2, s3):
        _body(aid, pid, nid, u, it, out, aidx, pidx, nidx, a_v,
              (b0, b1, b2), pd_s, out_v, (s0, s1, s2, s3))

    fn = pl.kernel(
        body,
        mesh=mesh,
        out_type=jax.ShapeDtypeStruct((_BATCH, _OPAD), jnp.float32),
        scratch_types=[
            pltpu.VMEM((_BPW,), jnp.int32),
            pltpu.VMEM((_BPW,), jnp.int32),
            pltpu.VMEM((_BPW * _NNEG,), jnp.int32),
            pltpu.VMEM((_BPW, _D), jnp.float32),
            pltpu.VMEM((_ROWS, _D), jnp.float32),
            pltpu.VMEM((_ROWS, _D), jnp.float32),
            pltpu.VMEM((_ROWS, _D), jnp.float32),
            pltpu.SMEM((_BPW,), jnp.float32),
            pltpu.VMEM((_BPW, _OPAD), jnp.float32),
            pltpu.SemaphoreType.DMA,
            pltpu.SemaphoreType.DMA,
            pltpu.SemaphoreType.DMA,
            pltpu.SemaphoreType.DMA,
        ],
    )
    return fn(anchor_ids, pos_ids, negf_ids, users, items)


def kernel(anchor_ids, pos_ids, neg_ids, users, items):
    negf = neg_ids.reshape(-1)
    out = _pairwise_sc(anchor_ids, pos_ids, negf, users, items)
    return out[:, :_NNEG]


# speedup vs baseline: 1.9752x; 1.9752x over previous
import jax
import jax.numpy as jnp
from jax import lax
from jax.experimental import pallas as pl
from jax.experimental.pallas import tpu as pltpu
from jax.experimental.pallas import tpu_sc as plsc

_INFO = plsc.get_sparse_core_info()
_NC = _INFO.num_cores
_NS = _INFO.num_subcores
_L = _INFO.num_lanes
_NW = _NC * _NS

_BATCH = 4096
_NNEG = 50
_D = 128
_CH = _D // _L
_BPW = _BATCH // _NW
_NG = _BPW // _L
_JC = 2
_NCHUNK = _NNEG // _JC
_NBUF = 3


def _body(aid_hbm, pid_hbm, nidT_hbm, users_hbm, items_hbm, out_hbm,
          aidx_v, pidx_v, nidx_v, a_v, bufs, pd_v, outc, sems):
    wid = lax.axis_index("s") * _NC + lax.axis_index("c")
    base = wid * _BPW

    pltpu.sync_copy(aid_hbm.at[pl.ds(base, _BPW)], aidx_v)
    pltpu.sync_copy(pid_hbm.at[pl.ds(base, _BPW)], pidx_v)
    pltpu.sync_copy(nidT_hbm.at[:, pl.ds(base, _BPW)], nidx_v)

    def fire(ch, b):
        for jl in range(_JC):
            pltpu.async_copy(
                items_hbm.at[nidx_v.at[ch * _JC + jl]],
                bufs[b].at[pl.ds(jl * _BPW, _BPW)], sems[b])

    def drain(b):
        for jl in range(_JC):
            pltpu.make_async_copy(
                items_hbm.at[nidx_v.at[0]],
                bufs[b].at[pl.ds(jl * _BPW, _BPW)], sems[b]).wait()

    cp_a = pltpu.async_copy(users_hbm.at[aidx_v], a_v, sems[_NBUF])
    cp_p = pltpu.async_copy(
        items_hbm.at[pidx_v], bufs[2].at[pl.ds(0, _BPW)], sems[2])
    fire(0, 0)
    fire(1, 1)

    i0 = lax.iota(jnp.int32, _L)
    perms = [i0 ^ 8, i0 ^ 4, i0 ^ 2, i0 ^ 1]

    def butterfly(acc):
        for p in perms:
            acc = acc + acc.at[p].get(mode="promise_in_bounds")
        return acc

    cp_a.wait()
    cp_p.wait()

    def pg(g, carry):
        res = jnp.zeros((_L,), jnp.float32)
        for rl in range(_L):
            r = g * _L + rl
            acc = jnp.zeros((_L,), jnp.float32)
            for c in range(_CH):
                a = a_v[r, pl.ds(c * _L, _L)]
                b = bufs[2][r, pl.ds(c * _L, _L)]
                d = a - b
                acc = acc + d * d
            res = jnp.where(i0 == rl, butterfly(acc), res)
        pd_v[pl.ds(g * _L, _L)] = res
        return carry

    lax.fori_loop(0, _NG, pg, 0)
    fire(2, 2)

    def chunk_compute(buf, ch, oc):
        def ng(g, c2):
            res = [jnp.zeros((_L,), jnp.float32) for _ in range(_JC)]
            for rl in range(_L):
                r = g * _L + rl
                a = [a_v[r, pl.ds(c * _L, _L)] for c in range(_CH)]
                for jl in range(_JC):
                    acc = jnp.zeros((_L,), jnp.float32)
                    for c in range(_CH):
                        b = buf[jl * _BPW + r, pl.ds(c * _L, _L)]
                        d = a[c] - b
                        acc = acc + d * d
                    res[jl] = jnp.where(i0 == rl, butterfly(acc), res[jl])
            pd = pd_v[pl.ds(g * _L, _L)]
            for jl in range(_JC):
                oc[jl, pl.ds(g * _L, _L)] = pd - res[jl]
            return c2

        lax.fori_loop(0, _NG, ng, 0)
        pltpu.async_copy(
            oc, out_hbm.at[pl.ds(ch * _JC, _JC), pl.ds(base, _BPW)],
            sems[_NBUF + 1])

    def drain_out(ch):
        pltpu.make_async_copy(
            outc[0], out_hbm.at[pl.ds(0, _JC), pl.ds(base, _BPW)],
            sems[_NBUF + 1]).wait()

    noc = len(outc)

    def trio(m, carry):
        ch0 = m * _NBUF
        for b in range(_NBUF):
            ch = ch0 + b

            @pl.when(ch >= noc)
            def _():
                drain_out(ch - noc)

            drain(b)
            chunk_compute(bufs[b], ch, outc[b % noc])

            @pl.when(ch + _NBUF < _NCHUNK)
            def _():
                fire(ch + _NBUF, b)

        return carry

    lax.fori_loop(0, _NCHUNK // _NBUF, trio, 0)
    for ch in range(_NCHUNK // _NBUF * _NBUF, _NCHUNK):
        bb = ch % _NBUF
        drain_out(ch - noc)
        drain(bb)
        chunk_compute(bufs[bb], ch, outc[ch % noc])
    for _k in range(noc):
        drain_out(0)


@jax.jit
def _pairwise_sc(anchor_ids, pos_ids, negT_ids, users, items):
    mesh = plsc.VectorSubcoreMesh(core_axis_name="c", subcore_axis_name="s")

    def body(aid, pid, nid, u, it, out, aidx, pidx, nidx, a_v,
             b0, b1, b2, pd_v, oc0, oc1, oc2, s0, s1, s2, s3, s4):
        _body(aid, pid, nid, u, it, out, aidx, pidx, nidx, a_v,
              (b0, b1, b2), pd_v, (oc0, oc1, oc2), (s0, s1, s2, s3, s4))

    fn = pl.kernel(
        body,
        mesh=mesh,
        out_type=jax.ShapeDtypeStruct((_NNEG, _BATCH), jnp.float32),
        scratch_types=[
            pltpu.VMEM((_BPW,), jnp.int32),
            pltpu.VMEM((_BPW,), jnp.int32),
            pltpu.VMEM((_NNEG, _BPW), jnp.int32),
            pltpu.VMEM((_BPW, _D), jnp.float32),
            pltpu.VMEM((_JC * _BPW, _D), jnp.float32),
            pltpu.VMEM((_JC * _BPW, _D), jnp.float32),
            pltpu.VMEM((_JC * _BPW, _D), jnp.float32),
            pltpu.VMEM((_BPW,), jnp.float32),
            pltpu.VMEM((_JC, _BPW), jnp.float32),
            pltpu.VMEM((_JC, _BPW), jnp.float32),
            pltpu.VMEM((_JC, _BPW), jnp.float32),
            pltpu.SemaphoreType.DMA,
            pltpu.SemaphoreType.DMA,
            pltpu.SemaphoreType.DMA,
            pltpu.SemaphoreType.DMA,
            pltpu.SemaphoreType.DMA,
        ],
    )
    return fn(anchor_ids, pos_ids, negT_ids, users, items)


def kernel(anchor_ids, pos_ids, neg_ids, users, items):
    negT = neg_ids.T
    outT = _pairwise_sc(anchor_ids, pos_ids, negT, users, items)
    return outT.T
